# baseline (device time: 473888 ns/iter reference)
import jax
import jax.numpy as jnp
from jax import lax
from jax.experimental import pallas as pl
from jax.experimental.pallas import tpu as pltpu

P = 16
BC = 32
SLOTS = 4


def kernel(x):
    m, n = x.shape
    qr = m // 4
    pr = qr // P
    br = m // BC

    def body(x_ref, o_ref, stage, obuf, sendbuf, in_sems, out_sems,
             yq_s, yq_r, xf_s, xf_r, zf_s, zf_r, xd_s, xd_r, zd_s, zd_r):
        my_x = lax.axis_index("x")
        my_y = lax.axis_index("y")
        my_z = lax.axis_index("z")
        y_nbr = (my_x, 1 - my_y, my_z)
        x_nbr = (1 - my_x, my_y, my_z)
        z_nbr = (my_x, my_y, 1 - my_z)

        barrier_sem = pltpu.get_barrier_semaphore()
        for nbr in (y_nbr, x_nbr, z_nbr):
            pl.semaphore_signal(
                barrier_sem, inc=1, device_id=nbr,
                device_id_type=pl.DeviceIdType.MESH,
            )
        pl.semaphore_wait(barrier_sem, 3)

        qi = my_x * 2 + my_z
        qx = (1 - my_x) * 2 + my_z
        qz = my_x * 2 + (1 - my_z)
        half = (1 - my_y) * m
        fq = half + qi * qr
        fqx = half + qx * qr
        fqz = half + qz * qr

        q_ins = [None] * P
        yq = []
        for i in range(min(SLOTS, P)):
            q_ins[i] = pltpu.make_async_copy(
                x_ref.at[pl.ds(qi * qr + i * pr, pr), :],
                stage.at[i % SLOTS, pl.ds(0, pr), :], in_sems.at[i % SLOTS])
            q_ins[i].start()
        for i in range(P):
            q_ins[i].wait()
            sendbuf[i * pr:(i + 1) * pr, :] = (
                stage[i % SLOTS, :pr, :].astype(jnp.bfloat16))
            if i + SLOTS < P:
                j = i + SLOTS
                q_ins[j] = pltpu.make_async_copy(
                    x_ref.at[pl.ds(qi * qr + j * pr, pr), :],
                    stage.at[j % SLOTS, pl.ds(0, pr), :],
                    in_sems.at[j % SLOTS])
                q_ins[j].start()
            r = pltpu.make_async_remote_copy(
                src_ref=sendbuf.at[pl.ds(i * pr, pr), :],
                dst_ref=o_ref.at[pl.ds(my_y * m + qi * qr + i * pr, pr), :],
                send_sem=yq_s.at[i], recv_sem=yq_r.at[i],
                device_id=y_nbr, device_id_type=pl.DeviceIdType.MESH,
            )
            r.start()
            yq.append(r)

        b_ins = [None] * BC
        b_outs = [None] * BC
        for k in range(min(SLOTS, BC)):
            b_ins[k] = pltpu.make_async_copy(
                x_ref.at[pl.ds(k * br, br), :], stage.at[k % SLOTS],
                in_sems.at[k % SLOTS])
            b_ins[k].start()

        def bounce_chunk(k):
            s = k % SLOTS
            b_ins[k].wait()
            if k >= SLOTS:
                b_outs[k - SLOTS].wait()
            obuf[s] = stage[s].astype(jnp.bfloat16)
            b_outs[k] = pltpu.make_async_copy(
                obuf.at[s], o_ref.at[pl.ds(my_y * m + k * br, br), :],
                out_sems.at[s])
            b_outs[k].start()
            if k + SLOTS < BC:
                b_ins[k + SLOTS] = pltpu.make_async_copy(
                    x_ref.at[pl.ds((k + SLOTS) * br, br), :], stage.at[s],
                    in_sems.at[s])
                b_ins[k + SLOTS].start()

        xf, zf, xd, zd = [], [], [], []
        for i in range(P):
            yq[i].wait_recv()
            rx = pltpu.make_async_remote_copy(
                src_ref=o_ref.at[pl.ds(fq + i * pr, pr), :],
                dst_ref=o_ref.at[pl.ds(fq + i * pr, pr), :],
                send_sem=xf_s.at[i], recv_sem=xf_r.at[i],
                device_id=x_nbr, device_id_type=pl.DeviceIdType.MESH,
            )
            rx.start()
            xf.append(rx)
            rz = pltpu.make_async_remote_copy(
                src_ref=o_ref.at[pl.ds(fq + i * pr, pr), :],
                dst_ref=o_ref.at[pl.ds(fq + i * pr, pr), :],
                send_sem=zf_s.at[i], recv_sem=zf_r.at[i],
                device_id=z_nbr, device_id_type=pl.DeviceIdType.MESH,
            )
            rz.start()
            zf.append(rz)
            if i < P // 2:
                zf[i].wait_recv()
                r = pltpu.make_async_remote_copy(
                    src_ref=o_ref.at[pl.ds(fqz + i * pr, pr), :],
                    dst_ref=o_ref.at[pl.ds(fqz + i * pr, pr), :],
                    send_sem=xd_s.at[i], recv_sem=xd_r.at[i],
                    device_id=x_nbr, device_id_type=pl.DeviceIdType.MESH,
                )
                r.start()
                xd.append(r)
            else:
                xf[i].wait_recv()
                r = pltpu.make_async_remote_copy(
                    src_ref=o_ref.at[pl.ds(fqx + i * pr, pr), :],
                    dst_ref=o_ref.at[pl.ds(fqx + i * pr, pr), :],
                    send_sem=zd_s.at[i - P // 2],
                    recv_sem=zd_r.at[i - P // 2],
                    device_id=z_nbr, device_id_type=pl.DeviceIdType.MESH,
                )
                r.start()
                zd.append(r)
            bounce_chunk(2 * i)
            bounce_chunk(2 * i + 1)

        for i in range(P // 2, P):
            zf[i].wait_recv()
        for i in range(P // 2):
            xf[i].wait_recv()
        for r in xd + zd:
            r.wait_recv()
            r.wait_send()
        for i in range(P):
            yq[i].wait_send()
            xf[i].wait_send()
            zf[i].wait_send()
        for i in range(max(0, BC - SLOTS), BC):
            b_outs[i].wait()

    out_shape = jax.ShapeDtypeStruct((2 * m, n), jnp.bfloat16)
    return pl.pallas_call(
        body,
        out_shape=out_shape,
        in_specs=[pl.BlockSpec(memory_space=pl.ANY)],
        out_specs=pl.BlockSpec(memory_space=pl.ANY),
        scratch_shapes=[
            pltpu.VMEM((SLOTS, br, n), jnp.float32),
            pltpu.VMEM((SLOTS, br, n), jnp.bfloat16),
            pltpu.VMEM((qr, n), jnp.bfloat16),
            pltpu.SemaphoreType.DMA((SLOTS,)),
            pltpu.SemaphoreType.DMA((SLOTS,)),
            pltpu.SemaphoreType.DMA((P,)),
            pltpu.SemaphoreType.DMA((P,)),
            pltpu.SemaphoreType.DMA((P,)),
            pltpu.SemaphoreType.DMA((P,)),
            pltpu.SemaphoreType.DMA((P,)),
            pltpu.SemaphoreType.DMA((P,)),
            pltpu.SemaphoreType.DMA((P // 2,)),
            pltpu.SemaphoreType.DMA((P // 2,)),
            pltpu.SemaphoreType.DMA((P // 2,)),
            pltpu.SemaphoreType.DMA((P // 2,)),
        ],
        compiler_params=pltpu.CompilerParams(
            collective_id=0, vmem_limit_bytes=64 * 1024 * 1024),
    )(x)


# device time: 437866 ns/iter; 1.0823x vs baseline; 1.0823x over previous
import jax
import jax.numpy as jnp
from jax import lax
from jax.experimental import pallas as pl
from jax.experimental.pallas import tpu as pltpu

P = 16
BC = 32
SLOTS = 4


def kernel(x):
    m, n = x.shape
    qr = m // 4
    pr = qr // P
    br = m // BC

    def body(x_ref, o_ref, stage, obuf, sendbuf, in_sems, out_sems,
             yq_s, yq_r, xf_s, xf_r, zf_s, zf_r, xd_s, xd_r, zd_s, zd_r):
        my_x = lax.axis_index("x")
        my_y = lax.axis_index("y")
        my_z = lax.axis_index("z")
        y_nbr = (my_x, 1 - my_y, my_z)
        x_nbr = (1 - my_x, my_y, my_z)
        z_nbr = (my_x, my_y, 1 - my_z)

        barrier_sem = pltpu.get_barrier_semaphore()
        for nbr in (y_nbr, x_nbr, z_nbr):
            pl.semaphore_signal(
                barrier_sem, inc=1, device_id=nbr,
                device_id_type=pl.DeviceIdType.MESH,
            )
        pl.semaphore_wait(barrier_sem, 3)

        qi = my_x * 2 + my_z
        qx = (1 - my_x) * 2 + my_z
        qz = my_x * 2 + (1 - my_z)
        half = (1 - my_y) * m
        fq = half + qi * qr
        fqx = half + qx * qr
        fqz = half + qz * qr

        q_ins = [None] * P
        yq = []
        for i in range(min(SLOTS, P)):
            q_ins[i] = pltpu.make_async_copy(
                x_ref.at[pl.ds(qi * qr + i * pr, pr), :],
                stage.at[i % SLOTS, pl.ds(0, pr), :], in_sems.at[i % SLOTS])
            q_ins[i].start()
        for i in range(P):
            q_ins[i].wait()
            sendbuf[i * pr:(i + 1) * pr, :] = (
                stage[i % SLOTS, :pr, :].astype(jnp.bfloat16))
            if i + SLOTS < P:
                j = i + SLOTS
                q_ins[j] = pltpu.make_async_copy(
                    x_ref.at[pl.ds(qi * qr + j * pr, pr), :],
                    stage.at[j % SLOTS, pl.ds(0, pr), :],
                    in_sems.at[j % SLOTS])
                q_ins[j].start()
            r = pltpu.make_async_remote_copy(
                src_ref=sendbuf.at[pl.ds(i * pr, pr), :],
                dst_ref=o_ref.at[pl.ds(my_y * m + qi * qr + i * pr, pr), :],
                send_sem=yq_s.at[i], recv_sem=yq_r.at[i],
                device_id=y_nbr, device_id_type=pl.DeviceIdType.MESH,
            )
            r.start()
            yq.append(r)

        b_ins = [None] * BC
        b_outs = [None] * BC
        b_ins[0] = pltpu.make_async_copy(
            x_ref.at[pl.ds(0, br), :], stage.at[0], in_sems.at[0])
        b_ins[0].start()
        for i in range(BC):
            s = i % SLOTS
            if i + 1 < BC:
                s1 = (i + 1) % SLOTS
                b_ins[i + 1] = pltpu.make_async_copy(
                    x_ref.at[pl.ds((i + 1) * br, br), :], stage.at[s1],
                    in_sems.at[s1])
                b_ins[i + 1].start()
            b_ins[i].wait()
            if i >= SLOTS:
                b_outs[i - SLOTS].wait()
            obuf[s] = stage[s].astype(jnp.bfloat16)
            b_outs[i] = pltpu.make_async_copy(
                obuf.at[s], o_ref.at[pl.ds(my_y * m + i * br, br), :],
                out_sems.at[s])
            b_outs[i].start()

        xf, zf = [], []
        for i in range(P):
            yq[i].wait_recv()
            rx = pltpu.make_async_remote_copy(
                src_ref=o_ref.at[pl.ds(fq + i * pr, pr), :],
                dst_ref=o_ref.at[pl.ds(fq + i * pr, pr), :],
                send_sem=xf_s.at[i], recv_sem=xf_r.at[i],
                device_id=x_nbr, device_id_type=pl.DeviceIdType.MESH,
            )
            rx.start()
            xf.append(rx)
            rz = pltpu.make_async_remote_copy(
                src_ref=o_ref.at[pl.ds(fq + i * pr, pr), :],
                dst_ref=o_ref.at[pl.ds(fq + i * pr, pr), :],
                send_sem=zf_s.at[i], recv_sem=zf_r.at[i],
                device_id=z_nbr, device_id_type=pl.DeviceIdType.MESH,
            )
            rz.start()
            zf.append(rz)

        xd = []
        for i in range(P // 2):
            zf[i].wait_recv()
            r = pltpu.make_async_remote_copy(
                src_ref=o_ref.at[pl.ds(fqz + i * pr, pr), :],
                dst_ref=o_ref.at[pl.ds(fqz + i * pr, pr), :],
                send_sem=xd_s.at[i], recv_sem=xd_r.at[i],
                device_id=x_nbr, device_id_type=pl.DeviceIdType.MESH,
            )
            r.start()
            xd.append(r)
        zd = []
        for i in range(P // 2, P):
            xf[i].wait_recv()
            r = pltpu.make_async_remote_copy(
                src_ref=o_ref.at[pl.ds(fqx + i * pr, pr), :],
                dst_ref=o_ref.at[pl.ds(fqx + i * pr, pr), :],
                send_sem=zd_s.at[i - P // 2], recv_sem=zd_r.at[i - P // 2],
                device_id=z_nbr, device_id_type=pl.DeviceIdType.MESH,
            )
            r.start()
            zd.append(r)

        for i in range(P // 2, P):
            zf[i].wait_recv()
        for i in range(P // 2):
            xf[i].wait_recv()
        for r in xd + zd:
            r.wait_recv()
            r.wait_send()
        for i in range(P):
            yq[i].wait_send()
            xf[i].wait_send()
            zf[i].wait_send()
        for i in range(max(0, BC - SLOTS), BC):
            b_outs[i].wait()

    out_shape = jax.ShapeDtypeStruct((2 * m, n), jnp.bfloat16)
    return pl.pallas_call(
        body,
        out_shape=out_shape,
        in_specs=[pl.BlockSpec(memory_space=pl.ANY)],
        out_specs=pl.BlockSpec(memory_space=pl.ANY),
        scratch_shapes=[
            pltpu.VMEM((SLOTS, br, n), jnp.float32),
            pltpu.VMEM((SLOTS, br, n), jnp.bfloat16),
            pltpu.VMEM((qr, n), jnp.bfloat16),
            pltpu.SemaphoreType.DMA((SLOTS,)),
            pltpu.SemaphoreType.DMA((SLOTS,)),
            pltpu.SemaphoreType.DMA((P,)),
            pltpu.SemaphoreType.DMA((P,)),
            pltpu.SemaphoreType.DMA((P,)),
            pltpu.SemaphoreType.DMA((P,)),
            pltpu.SemaphoreType.DMA((P,)),
            pltpu.SemaphoreType.DMA((P,)),
            pltpu.SemaphoreType.DMA((P // 2,)),
            pltpu.SemaphoreType.DMA((P // 2,)),
            pltpu.SemaphoreType.DMA((P // 2,)),
            pltpu.SemaphoreType.DMA((P // 2,)),
        ],
        compiler_params=pltpu.CompilerParams(
            collective_id=0, vmem_limit_bytes=64 * 1024 * 1024),
    )(x)
